# Initial kernel scaffold; baseline (speedup 1.0000x reference)
import functools

import jax
import jax.numpy as jnp
from jax.experimental import pallas as pl
from jax.experimental.pallas import tpu as pltpu
from jax.experimental.pallas import tpu_sc as plsc

_N = 10000
_BQ = 1000
_BK = 1000
_NQ = _N // _BQ
_NK = _N // _BK
_W = 48
_GW = 80
_PREC = jax.lax.Precision.HIGHEST


def _nn_body(e1_ref, e2_ref, out_ref, min_ref, arg_ref):
    j = pl.program_id(1)

    @pl.when(j == 0)
    def _():
        min_ref[...] = jnp.full(min_ref.shape, jnp.inf, min_ref.dtype)
        arg_ref[...] = jnp.zeros(arg_ref.shape, arg_ref.dtype)

    q = e2_ref[...]
    b = e1_ref[...]
    s = jax.lax.dot_general(q, b, (((1,), (1,)), ((), ())),
                            precision=_PREC,
                            preferred_element_type=jnp.float32)
    q2 = jnp.sum(q * q, axis=1, keepdims=True)
    n1 = jnp.sum(b * b, axis=1)
    d = (q2 - 2.0 * s) + n1[None, :]
    m = jnp.min(d, axis=1)
    lane = jax.lax.broadcasted_iota(jnp.int32, d.shape, 1)
    a = jnp.min(jnp.where(d == m[:, None], lane, jnp.int32(2**30)), axis=1)
    a = a + j * _BK
    better = m < min_ref[0, :]
    arg_ref[0, :] = jnp.where(better, a, arg_ref[0, :])
    min_ref[0, :] = jnp.where(better, m, min_ref[0, :])

    @pl.when(j == _NK - 1)
    def _():
        out_ref[0, 0, :] = arg_ref[0, :]


def _nn(emb1, emb2):
    d = emb1.shape[1]
    out = pl.pallas_call(
        _nn_body,
        grid=(_NQ, _NK),
        in_specs=[pl.BlockSpec((_BK, d), lambda i, j: (j, 0)),
                  pl.BlockSpec((_BQ, d), lambda i, j: (i, 0))],
        out_specs=pl.BlockSpec((1, 1, _BQ), lambda i, j: (i, 0, 0)),
        out_shape=jax.ShapeDtypeStruct((_NQ, 1, _BQ), jnp.int32),
        scratch_shapes=[pltpu.VMEM((1, _BQ), jnp.float32),
                        pltpu.VMEM((1, _BQ), jnp.int32)],
    )(emb1, emb2)
    return out.reshape(_N)


def _gather(data, idx):
    w = data.shape[1]
    idx2 = idx.reshape(1, _N)
    mesh = plsc.VectorSubcoreMesh(core_axis_name="c", subcore_axis_name="s")

    @functools.partial(pl.kernel,
                       out_type=jax.ShapeDtypeStruct((_N, w), data.dtype),
                       mesh=mesh)
    def run(x_hbm, i_hbm, o_hbm):
        def body(i_vmem, o_vmem):
            pltpu.sync_copy(x_hbm.at[i_vmem.at[0]], o_vmem)

        pltpu.emit_pipeline(
            body,
            grid=(_N // _GW,),
            in_specs=[pl.BlockSpec((1, _GW), lambda i: (0, i))],
            out_specs=[pl.BlockSpec((_GW, w), lambda i: (i, 0))],
            core_axis_name=("c", "s"),
            dimension_semantics=(pltpu.PARALLEL,),
        )(i_hbm, o_hbm)

    return run(data, idx2)


def _c12_body(e2_ref, g_ref, m_ref, out_ref):
    i = pl.program_id(0)
    mg = m_ref[0, 0, :][:, None] * g_ref[...]
    part = jax.lax.dot_general(e2_ref[...], mg, (((0,), (0,)), ((), ())),
                               precision=_PREC,
                               preferred_element_type=jnp.float32)

    @pl.when(i == 0)
    def _():
        out_ref[...] = part

    @pl.when(i > 0)
    def _():
        out_ref[...] += part


def _c12(e2w, g, mass3):
    return pl.pallas_call(
        _c12_body,
        grid=(_NQ,),
        in_specs=[pl.BlockSpec((_BQ, _W), lambda i: (i, 0)),
                  pl.BlockSpec((_BQ, _W), lambda i: (i, 0)),
                  pl.BlockSpec((1, 1, _BQ), lambda i: (i, 0, 0))],
        out_specs=pl.BlockSpec((_W, _W), lambda i: (0, 0)),
        out_shape=jax.ShapeDtypeStruct((_W, _W), jnp.float32),
    )(e2w, g, mass3)


def _emb_body(k2, e1_ref, e2_ref, c_ref, o1_ref, o2_ref):
    p = jax.lax.dot_general(e1_ref[...], c_ref[...], (((1,), (1,)), ((), ())),
                            precision=_PREC,
                            preferred_element_type=jnp.float32)
    o1_ref[...] = p / k2
    o2_ref[...] = e2_ref[...] / k2


def _emb(e1p, e2p, cpad, k2):
    return pl.pallas_call(
        functools.partial(_emb_body, jnp.float32(k2)),
        grid=(_NQ,),
        in_specs=[pl.BlockSpec((_BQ, 32), lambda i: (i, 0)),
                  pl.BlockSpec((_BQ, 32), lambda i: (i, 0)),
                  pl.BlockSpec((32, 32), lambda i: (0, 0))],
        out_specs=[pl.BlockSpec((_BQ, 32), lambda i: (i, 0)),
                   pl.BlockSpec((_BQ, 32), lambda i: (i, 0))],
        out_shape=[jax.ShapeDtypeStruct((_N, 32), jnp.float32),
                   jax.ShapeDtypeStruct((_N, 32), jnp.float32)],
    )(e1p, e2p, cpad)


def kernel(F1, F2, evects1, evects2, mass2, return_T21):
    e1w = evects1[:, :_W]
    e2w = evects2[:, :_W]
    e1_32 = evects1[:, :32]
    mass3 = mass2.reshape(_NQ, 1, _BQ)

    idx = _nn(F1, F2)
    g = _gather(e1w, idx)
    c_full = _c12(e2w, g, mass3)
    c12 = c_full[:20, :20]

    for k_prev, k_curr in ((20, 30), (30, 40)):
        cpad = jnp.pad(c12, ((0, 32 - k_prev), (0, 32 - k_prev)))
        e2p = jnp.pad(evects2[:, :k_prev], ((0, 0), (0, 32 - k_prev)))
        emb1, emb2 = _emb(e1_32, e2p, cpad, float(k_prev))
        idx = _nn(emb1, emb2)
        g = _gather(e1w, idx)
        c_full = _c12(e2w, g, mass3)
        c12 = c_full[:k_curr, :k_curr]

    return c12


# speedup vs baseline: 78.0541x; 78.0541x over previous
import functools

import jax
import jax.numpy as jnp
from jax.experimental import pallas as pl
from jax.experimental.pallas import tpu as pltpu
from jax.experimental.pallas import tpu_sc as plsc

_N = 10000
_BQ = 1000
_BK = 1000
_NQ = _N // _BQ
_NK = _N // _BK
_GW = 128
_NP = 10240


def _rowsq(x):
    t = x * x
    w = t.shape[1]
    if w % 8 != 0 or w <= 8:
        return jnp.sum(t, axis=1, keepdims=True)
    acc = t[:, 0:8]
    for c in range(1, w // 8):
        acc = acc + t[:, 8 * c:8 * c + 8]
    w = 8
    while w > 1:
        w //= 2
        acc = acc[:, :w] + acc[:, w:]
    return acc


def _nn_body(e1_ref, e2_ref, out_ref, min_ref, arg_ref):
    j = pl.program_id(1)

    @pl.when(j == 0)
    def _():
        min_ref[...] = jnp.full(min_ref.shape, jnp.inf, min_ref.dtype)
        arg_ref[...] = jnp.zeros(arg_ref.shape, arg_ref.dtype)

    q = e2_ref[...]
    b = e1_ref[...]
    s = jax.lax.dot_general(q.astype(jnp.bfloat16), b.astype(jnp.bfloat16),
                            (((1,), (1,)), ((), ())),
                            preferred_element_type=jnp.float32)
    q2 = _rowsq(q)
    n1 = _rowsq(b)[:, 0]
    d = (q2 - 2.0 * s) + n1[None, :]
    m = jnp.min(d, axis=1)
    lane = jax.lax.broadcasted_iota(jnp.int32, d.shape, 1)
    a = jnp.min(jnp.where(d == m[:, None], lane, jnp.int32(2**30)), axis=1)
    a = a + j * _BK
    better = m < min_ref[0, :]
    arg_ref[0, :] = jnp.where(better, a, arg_ref[0, :])
    min_ref[0, :] = jnp.where(better, m, min_ref[0, :])

    @pl.when(j == _NK - 1)
    def _():
        out_ref[0, 0, :] = arg_ref[0, :]


def _nn(emb1, emb2):
    d = emb1.shape[1]
    out = pl.pallas_call(
        _nn_body,
        grid=(_NQ, _NK),
        in_specs=[pl.BlockSpec((_BK, d), lambda i, j: (j, 0)),
                  pl.BlockSpec((_BQ, d), lambda i, j: (i, 0))],
        out_specs=pl.BlockSpec((1, 1, _BQ), lambda i, j: (i, 0, 0)),
        out_shape=jax.ShapeDtypeStruct((_NQ, 1, _BQ), jnp.int32),
        scratch_shapes=[pltpu.VMEM((1, _BQ), jnp.float32),
                        pltpu.VMEM((1, _BQ), jnp.int32)],
    )(emb1, emb2)
    return out.reshape(_N)


def _gather(data, idx):
    w = data.shape[1]
    idx2 = jnp.pad(idx, (0, _NP - _N)).reshape(1, _NP)
    mesh = plsc.VectorSubcoreMesh(core_axis_name="c", subcore_axis_name="s")

    @functools.partial(pl.kernel,
                       out_type=jax.ShapeDtypeStruct((_NP, w), data.dtype),
                       mesh=mesh)
    def run(x_hbm, i_hbm, o_hbm):
        def body(i_vmem, o_vmem):
            pltpu.sync_copy(x_hbm.at[i_vmem.at[0]], o_vmem)

        pltpu.emit_pipeline(
            body,
            grid=(_NP // _GW,),
            in_specs=[pl.BlockSpec((1, _GW), lambda i: (0, i))],
            out_specs=[pl.BlockSpec((_GW, w), lambda i: (i, 0))],
            core_axis_name=("c", "s"),
            dimension_semantics=(pltpu.PARALLEL,),
        )(i_hbm, o_hbm)

    return run(data, idx2)[:_N]


def _c12_body(e2_ref, g_ref, m_ref, out_ref):
    mg = m_ref[...] * g_ref[...]
    out_ref[...] = jax.lax.dot_general(e2_ref[...], mg, (((0,), (0,)), ((), ())),
                                       preferred_element_type=jnp.float32)


def _c12(e2k, gk, mass1):
    k = e2k.shape[1]
    return pl.pallas_call(
        _c12_body,
        in_specs=[pl.BlockSpec((_N, k), lambda: (0, 0)),
                  pl.BlockSpec((_N, k), lambda: (0, 0)),
                  pl.BlockSpec((_N, 1), lambda: (0, 0))],
        out_specs=pl.BlockSpec((k, k), lambda: (0, 0)),
        out_shape=jax.ShapeDtypeStruct((k, k), jnp.float32),
    )(e2k, gk, mass1)


def _emb_body(k2, e1_ref, c_ref, o_ref):
    o_ref[...] = jax.lax.dot_general(e1_ref[...], c_ref[...],
                                     (((1,), (1,)), ((), ())),
                                     preferred_element_type=jnp.float32) / k2


def _emb(e1k, c12, k2):
    k = e1k.shape[1]
    return pl.pallas_call(
        functools.partial(_emb_body, float(k2)),
        in_specs=[pl.BlockSpec((_N, k), lambda: (0, 0)),
                  pl.BlockSpec((k, k), lambda: (0, 0))],
        out_specs=pl.BlockSpec((_N, k), lambda: (0, 0)),
        out_shape=jax.ShapeDtypeStruct((_N, k), jnp.float32),
    )(e1k, c12)


def kernel(F1, F2, evects1, evects2, mass2, return_T21):
    mass1 = mass2.reshape(_N, 1)

    dmat = (jnp.sum(F2 * F2, axis=1, keepdims=True)
            - 2.0 * F2 @ F1.T
            + jnp.sum(F1 * F1, axis=1)[None, :])
    idx = jnp.argmin(dmat, axis=1)
    g = _gather(evects1, idx)
    c12 = _c12(evects2[:, :20], g[:, :20], mass1)

    for k_prev, k_curr in ((20, 30), (30, 40)):
        emb1 = _emb(evects1[:, :k_prev], c12, float(k_prev))
        emb2 = evects2[:, :k_prev] / k_prev
        idx = _nn(emb1, emb2)
        g = _gather(evects1, idx)
        c12 = _c12(evects2[:, :k_curr], g[:, :k_curr], mass1)

    return c12
